# baseline (speedup 1.0000x reference)
import jax
import jax.numpy as jnp
from jax import lax
from jax.experimental import pallas as pl
from jax.experimental.pallas import tpu as pltpu

B, S, P = 2, 128, 256
D = 1024
H = 16
DH = D // H

G = 4
HPG = H // G
DG = D // G
RG = S // G

_XWT = (((1,), (1,)), ((), ()))


def _fused_body(shelf_ref, product_ref, supply_ref, wq_ref, bq_ref,
                wk_ref, bk_ref, wv_ref, bv_ref, wo_ref, bo_ref,
                out_ref, wo3_s):
    @pl.when((pl.program_id(0) == 0) & (pl.program_id(1) == 0))
    def _prep():
        wo3_s[...] = wo_ref[...].T.reshape(16, DH, D)

    x_s = shelf_ref[0]
    x_p = product_ref[0]

    q = lax.dot_general(x_s, wq_ref[...], _XWT,
                        preferred_element_type=jnp.float32) + bq_ref[...]
    k = lax.dot_general(x_p, wk_ref[...], _XWT,
                        preferred_element_type=jnp.float32) + bk_ref[...]
    v = lax.dot_general(x_p, wv_ref[...], _XWT,
                        preferred_element_type=jnp.float32) + bv_ref[...]

    q4 = q.reshape(S, HPG, DH)
    k4 = k.reshape(P, HPG, DH)
    v4 = v.reshape(P, HPG, DH)

    logits = lax.dot_general(
        q4, k4,
        dimension_numbers=(((2,), (2,)), ((1,), (1,))),
        preferred_element_type=jnp.float32,
    ) * (1.0 / (DH ** 0.5))

    mask = (supply_ref[0] > 0)[None, :, :]
    m = jnp.max(logits, axis=2, keepdims=True)
    e = jnp.where(mask, jnp.exp(logits - m), 0.0)
    den = jnp.sum(e, axis=2, keepdims=True)
    w = e / (den + 1e-9)

    attn = lax.dot_general(
        w, v4,
        dimension_numbers=(((2,), (0,)), ((0,), (1,))),
        preferred_element_type=jnp.float32,
    )

    a4 = attn.reshape(HPG, 8, 16, DH)
    t = lax.dot_general(
        a4, wo3_s[...],
        dimension_numbers=(((3,), (1,)), ((2,), (0,))),
        preferred_element_type=jnp.float32,
    )
    out_ref[0] = jnp.sum(t, axis=0).reshape(RG, D) + bo_ref[...]


@jax.jit
def kernel(shelf_embs, product_embs, supply, W_q, b_q, W_k, b_k, W_v, b_v, W_o, b_o):
    out = pl.pallas_call(
        _fused_body,
        grid=(G, B),
        in_specs=[
            pl.BlockSpec((1, S, D), lambda g, b: (b, 0, 0)),
            pl.BlockSpec((1, P, D), lambda g, b: (b, 0, 0)),
            pl.BlockSpec((1, S, P), lambda g, b: (b, 0, 0)),
            pl.BlockSpec((DG, D), lambda g, b: (g, 0)),
            pl.BlockSpec((DG,), lambda g, b: (g,)),
            pl.BlockSpec((DG, D), lambda g, b: (g, 0)),
            pl.BlockSpec((DG,), lambda g, b: (g,)),
            pl.BlockSpec((DG, D), lambda g, b: (g, 0)),
            pl.BlockSpec((DG,), lambda g, b: (g,)),
            pl.BlockSpec((D, D), lambda g, b: (0, 0)),
            pl.BlockSpec((D,), lambda g, b: (0,)),
        ],
        out_specs=pl.BlockSpec((1, RG, D), lambda g, b: (b, g, 0)),
        out_shape=jax.ShapeDtypeStruct((B, S, D), jnp.float32),
        scratch_shapes=[pltpu.VMEM((16, DH, D), jnp.float32)],
    )(shelf_embs, product_embs, supply, W_q, b_q, W_k, b_k, W_v, b_v, W_o, b_o)
    return out


# speedup vs baseline: 1.2370x; 1.2370x over previous
import jax
import jax.numpy as jnp
from jax import lax
from jax.experimental import pallas as pl
from jax.experimental.pallas import tpu as pltpu

B, S, P = 2, 128, 256
D = 1024
H = 16
DH = D // H

G = 2
HPG = H // G
DG = D // G
RG = S // G

_XWT = (((1,), (1,)), ((), ()))


def _fused_body(shelf_ref, product_ref, supply_ref, wq_ref, bq_ref,
                wk_ref, bk_ref, wv_ref, bv_ref, wo_ref, bo_ref,
                out_ref, wo3_s):
    @pl.when((pl.program_id(0) == 0) & (pl.program_id(1) == 0))
    def _prep():
        wo3_s[...] = wo_ref[...].T.reshape(16, DH, D)

    b = pl.program_id(1)
    x_s = shelf_ref[b]
    x_p = product_ref[b]

    q = lax.dot_general(x_s, wq_ref[...], _XWT,
                        preferred_element_type=jnp.float32) + bq_ref[...]
    k = lax.dot_general(x_p, wk_ref[...], _XWT,
                        preferred_element_type=jnp.float32) + bk_ref[...]
    v = lax.dot_general(x_p, wv_ref[...], _XWT,
                        preferred_element_type=jnp.float32) + bv_ref[...]

    q4 = q.reshape(S, HPG, DH)
    k4 = k.reshape(P, HPG, DH)
    v4 = v.reshape(P, HPG, DH)

    logits = lax.dot_general(
        q4, k4,
        dimension_numbers=(((2,), (2,)), ((1,), (1,))),
        preferred_element_type=jnp.float32,
    ) * (1.0 / (DH ** 0.5))

    mask = (supply_ref[b] > 0)[None, :, :]
    m = jnp.max(logits, axis=2, keepdims=True)
    e = jnp.where(mask, jnp.exp(logits - m), 0.0)
    den = jnp.sum(e, axis=2, keepdims=True)
    w = e / (den + 1e-9)

    attn = lax.dot_general(
        w, v4,
        dimension_numbers=(((2,), (0,)), ((0,), (1,))),
        preferred_element_type=jnp.float32,
    )

    a4 = attn.reshape(HPG, 8, 16, DH)
    t = lax.dot_general(
        a4, wo3_s[...],
        dimension_numbers=(((3,), (1,)), ((2,), (0,))),
        preferred_element_type=jnp.float32,
    )
    out_ref[0] = jnp.sum(t, axis=0).reshape(RG, D) + bo_ref[...]


@jax.jit
def kernel(shelf_embs, product_embs, supply, W_q, b_q, W_k, b_k, W_v, b_v, W_o, b_o):
    out = pl.pallas_call(
        _fused_body,
        grid=(G, B),
        in_specs=[
            pl.BlockSpec((B, S, D), lambda g, b: (0, 0, 0)),
            pl.BlockSpec((B, P, D), lambda g, b: (0, 0, 0)),
            pl.BlockSpec((B, S, P), lambda g, b: (0, 0, 0)),
            pl.BlockSpec((DG, D), lambda g, b: (g, 0)),
            pl.BlockSpec((DG,), lambda g, b: (g,)),
            pl.BlockSpec((DG, D), lambda g, b: (g, 0)),
            pl.BlockSpec((DG,), lambda g, b: (g,)),
            pl.BlockSpec((DG, D), lambda g, b: (g, 0)),
            pl.BlockSpec((DG,), lambda g, b: (g,)),
            pl.BlockSpec((D, D), lambda g, b: (0, 0)),
            pl.BlockSpec((D,), lambda g, b: (0,)),
        ],
        out_specs=pl.BlockSpec((1, RG, D), lambda g, b: (b, g, 0)),
        out_shape=jax.ShapeDtypeStruct((B, S, D), jnp.float32),
        scratch_shapes=[pltpu.VMEM((16, DH, D), jnp.float32)],
    )(shelf_embs, product_embs, supply, W_q, b_q, W_k, b_k, W_v, b_v, W_o, b_o)
    return out


# speedup vs baseline: 1.2914x; 1.0440x over previous
import jax
import jax.numpy as jnp
from jax import lax
from jax.experimental import pallas as pl
from jax.experimental.pallas import tpu as pltpu

B, S, P = 2, 128, 256
D = 1024
H = 16
DH = D // H

G = 2
HPG = H // G
DG = D // G
RG = S // G

_XWT = (((1,), (1,)), ((), ()))


def _fused_body(shelf_ref, product_ref, supply_ref, wq_ref, bq_ref,
                wk_ref, bk_ref, wv_ref, bv_ref, wo_ref, bo_ref,
                out_ref, wo3_s):
    @pl.when((pl.program_id(0) == 0) & (pl.program_id(1) == 0))
    def _prep():
        wo3_s[...] = wo_ref[...].T.reshape(16, DH, D)

    x_s = shelf_ref[0]
    x_p = product_ref[0]

    q = (lax.dot_general(x_s, wq_ref[...], _XWT,
                         preferred_element_type=jnp.float32)
         + bq_ref[...]) * (1.0 / (DH ** 0.5))
    k = lax.dot_general(x_p, wk_ref[...], _XWT,
                        preferred_element_type=jnp.float32) + bk_ref[...]
    v = lax.dot_general(x_p, wv_ref[...], _XWT,
                        preferred_element_type=jnp.float32) + bv_ref[...]

    q4 = q.reshape(S, HPG, DH)
    k4 = k.reshape(P, HPG, DH)
    v4 = v.reshape(P, HPG, DH)

    logits = lax.dot_general(
        q4, k4,
        dimension_numbers=(((2,), (2,)), ((1,), (1,))),
        preferred_element_type=jnp.float32,
    )

    mask = (supply_ref[0] > 0)[None, :, :]
    m = jnp.max(logits, axis=2, keepdims=True)
    e = jnp.where(mask, jnp.exp(logits - m), 0.0)
    den = jnp.sum(e, axis=2, keepdims=True)

    attn = lax.dot_general(
        e, v4,
        dimension_numbers=(((2,), (0,)), ((0,), (1,))),
        preferred_element_type=jnp.float32,
    ) / (den + 1e-9)

    a4 = attn.reshape(HPG, 8, 16, DH)
    t = lax.dot_general(
        a4, wo3_s[...],
        dimension_numbers=(((3,), (1,)), ((2,), (0,))),
        preferred_element_type=jnp.float32,
    )
    out_ref[0] = jnp.sum(t, axis=0).reshape(RG, D) + bo_ref[...]


@jax.jit
def kernel(shelf_embs, product_embs, supply, W_q, b_q, W_k, b_k, W_v, b_v, W_o, b_o):
    out = pl.pallas_call(
        _fused_body,
        grid=(G, B),
        in_specs=[
            pl.BlockSpec((1, S, D), lambda g, b: (b, 0, 0)),
            pl.BlockSpec((1, P, D), lambda g, b: (b, 0, 0)),
            pl.BlockSpec((1, S, P), lambda g, b: (b, 0, 0)),
            pl.BlockSpec((DG, D), lambda g, b: (g, 0)),
            pl.BlockSpec((DG,), lambda g, b: (g,)),
            pl.BlockSpec((DG, D), lambda g, b: (g, 0)),
            pl.BlockSpec((DG,), lambda g, b: (g,)),
            pl.BlockSpec((DG, D), lambda g, b: (g, 0)),
            pl.BlockSpec((DG,), lambda g, b: (g,)),
            pl.BlockSpec((D, D), lambda g, b: (0, 0)),
            pl.BlockSpec((D,), lambda g, b: (0,)),
        ],
        out_specs=pl.BlockSpec((1, RG, D), lambda g, b: (b, g, 0)),
        out_shape=jax.ShapeDtypeStruct((B, S, D), jnp.float32),
        scratch_shapes=[pltpu.VMEM((16, DH, D), jnp.float32)],
    )(shelf_embs, product_embs, supply, W_q, b_q, W_k, b_k, W_v, b_v, W_o, b_o)
    return out
